# baseline (speedup 1.0000x reference)
import functools

import jax
import jax.numpy as jnp
from jax import lax
from jax.experimental import pallas as pl
from jax.experimental.pallas import tpu as pltpu
from jax.experimental.pallas import tpu_sc as plsc


def _make_sc_kernel(B, D, NC, NS, L):
    NW = NC * NS
    b_per_w = B // NW
    CH = 128
    n_ch = b_per_w // CH

    mesh = plsc.VectorSubcoreMesh(
        core_axis_name="c", subcore_axis_name="s",
        num_cores=NC, num_subcores=NS)

    @functools.partial(
        pl.kernel,
        mesh=mesh,
        compiler_params=pltpu.CompilerParams(use_tc_tiling_on_sc=False),
        out_type=jax.ShapeDtypeStruct((NW, L), jnp.float32),
        scratch_types=[
            pltpu.VMEM((n_ch, CH), jnp.int32),
            pltpu.VMEM((b_per_w, D), jnp.float32),
            pltpu.VMEM((b_per_w, D), jnp.float32),
            pltpu.VMEM((L,), jnp.float32),
            pltpu.SemaphoreType.DMA,
        ],
    )
    def sc_kernel(centres_hbm, feat_hbm, lab_hbm, out_hbm,
                  idx_v, rows_v, feat_v, acc_v, sem):
        wid = lax.axis_index("s") * NC + lax.axis_index("c")
        pltpu.sync_copy(lab_hbm.at[wid], idx_v)
        copies = [
            pltpu.make_async_copy(
                centres_hbm.at[idx_v.at[j]],
                rows_v.at[pl.ds(j * CH, CH)],
                sem)
            for j in range(n_ch)
        ]
        for c in copies:
            c.start()
        pltpu.sync_copy(feat_hbm.at[wid], feat_v)
        for c in copies:
            c.wait()

        def body(i, acc):
            a0 = rows_v[i, pl.ds(0, L)] * feat_v[i, pl.ds(0, L)]
            a1 = rows_v[i, pl.ds(L, L)] * feat_v[i, pl.ds(L, L)]
            return acc + a0 + a1

        acc = lax.fori_loop(0, b_per_w, body,
                            jnp.zeros((L,), jnp.float32))
        acc_v[...] = acc
        pltpu.sync_copy(acc_v, out_hbm.at[wid])

    return sc_kernel


def kernel(centres, features, labels):
    B, D = features.shape
    info = plsc.get_sparse_core_info()
    NC, NS, L = info.num_cores, info.num_subcores, info.num_lanes
    NW = NC * NS
    b_per_w = B // NW
    lab = labels.astype(jnp.int32).reshape(NW, b_per_w // 128, 128)
    feat = features.reshape(NW, b_per_w, D)
    partials = _make_sc_kernel(B, D, NC, NS, L)(centres, feat, lab)
    return jnp.sum(partials)
